# baseline (speedup 1.0000x reference)
import functools

import jax
import jax.numpy as jnp
from jax import lax
from jax.experimental import pallas as pl
from jax.experimental.pallas import tpu as pltpu
from jax.experimental.pallas import tpu_sc as plsc

N_NODES = 10000
N_EDGES = 320000
NC = 2
NS = 16
NW = NC * NS
BATCH = 128
N_ROWS = N_EDGES // BATCH
ROWS_LO = N_ROWS // NW
N_HI = N_ROWS - ROWS_LO * NW
ROWS_MAX = ROWS_LO + 1
N_PAD = 10240
ROWS_PER_TILE = N_PAD // NS


def _worker_rows(wid):
    base = ROWS_LO * wid + jnp.minimum(wid, N_HI)
    count = ROWS_LO + (wid < N_HI).astype(jnp.int32)
    return base, count


def _make_agg(D, depth):
    mesh = plsc.VectorSubcoreMesh(core_axis_name="c", subcore_axis_name="s")

    @functools.partial(
        pl.kernel,
        mesh=mesh,
        compiler_params=pltpu.CompilerParams(use_tc_tiling_on_sc=False),
        out_type=jax.ShapeDtypeStruct((NC, N_PAD, D), jnp.float32),
        scratch_types=(
            [
                pltpu.VMEM((ROWS_MAX, BATCH), jnp.int32),
                pltpu.VMEM((ROWS_MAX, BATCH), jnp.int32),
            ]
            + [pltpu.VMEM((BATCH, D), jnp.float32)] * depth
            + [pltpu.VMEM_SHARED((N_PAD, D), jnp.float32)]
            + [pltpu.SemaphoreType.DMA] * depth
        ),
    )
    def agg(table_hbm, ei_hbm, zeros_hbm, out_hbm, *scratch):
        idx_s, idx_d = scratch[0], scratch[1]
        bufs = scratch[2:2 + depth]
        acc = scratch[2 + depth]
        sems = scratch[3 + depth:]
        cid = lax.axis_index("c")
        sid = lax.axis_index("s")
        wid = sid * NC + cid
        r0 = sid * ROWS_PER_TILE
        pltpu.sync_copy(zeros_hbm.at[pl.ds(r0, ROWS_PER_TILE)],
                        acc.at[pl.ds(r0, ROWS_PER_TILE)])
        base, count = _worker_rows(wid)
        pltpu.sync_copy(ei_hbm.at[0, pl.ds(base, ROWS_LO)],
                        idx_s.at[pl.ds(0, ROWS_LO)])
        pltpu.sync_copy(ei_hbm.at[1, pl.ds(base, ROWS_LO)],
                        idx_d.at[pl.ds(0, ROWS_LO)])

        @pl.when(count > ROWS_LO)
        def _():
            pltpu.sync_copy(ei_hbm.at[0, pl.ds(base + ROWS_LO, 1)],
                            idx_s.at[pl.ds(ROWS_LO, 1)])
            pltpu.sync_copy(ei_hbm.at[1, pl.ds(base + ROWS_LO, 1)],
                            idx_d.at[pl.ds(ROWS_LO, 1)])

        plsc.subcore_barrier()

        for j in range(depth):
            pltpu.async_copy(table_hbm.at[idx_s.at[j]], bufs[j], sems[j])

        def body(i, carry):
            e = depth * i
            for j in range(depth):
                pltpu.make_async_copy(
                    table_hbm.at[idx_s.at[0]], bufs[j], sems[j]).wait()
                pltpu.sync_copy(bufs[j], acc.at[idx_d.at[e + j]], add=True)
                nxt = lax.rem(e + j + depth, count)
                pltpu.async_copy(table_hbm.at[idx_s.at[nxt]], bufs[j], sems[j])
            return carry

        lax.fori_loop(0, ROWS_LO // depth, body, 0)

        e0 = (ROWS_LO // depth) * depth
        for j in range(depth):
            pltpu.make_async_copy(
                table_hbm.at[idx_s.at[0]], bufs[j], sems[j]).wait()
            e = e0 + j
            if e < ROWS_LO:
                pltpu.sync_copy(bufs[j], acc.at[idx_d.at[e]], add=True)
            elif e == ROWS_LO:
                @pl.when(count > ROWS_LO)
                def _(buf=bufs[j]):
                    pltpu.sync_copy(buf, acc.at[idx_d.at[ROWS_LO]], add=True)

        plsc.subcore_barrier()
        pltpu.sync_copy(acc.at[pl.ds(r0, ROWS_PER_TILE)],
                        out_hbm.at[cid, pl.ds(r0, ROWS_PER_TILE)])

    return agg


_agg16 = _make_agg(16, 8)
_agg64 = _make_agg(64, 6)

_DEG_D = 16


def _make_deg():
    mesh = plsc.VectorSubcoreMesh(core_axis_name="c", subcore_axis_name="s")

    @functools.partial(
        pl.kernel,
        mesh=mesh,
        compiler_params=pltpu.CompilerParams(use_tc_tiling_on_sc=False),
        out_type=jax.ShapeDtypeStruct((NC, N_PAD, _DEG_D), jnp.float32),
        scratch_types=[
            pltpu.VMEM((ROWS_MAX, BATCH), jnp.int32),
            pltpu.VMEM((BATCH, _DEG_D), jnp.float32),
            pltpu.VMEM_SHARED((N_PAD, _DEG_D), jnp.float32),
        ],
    )
    def deg(ones_hbm, ei_hbm, zeros_hbm, out_hbm, idx_d, rows, acc):
        cid = lax.axis_index("c")
        sid = lax.axis_index("s")
        wid = sid * NC + cid
        r0 = sid * ROWS_PER_TILE
        pltpu.sync_copy(zeros_hbm.at[pl.ds(r0, ROWS_PER_TILE)],
                        acc.at[pl.ds(r0, ROWS_PER_TILE)])
        base, count = _worker_rows(wid)
        pltpu.sync_copy(ei_hbm.at[1, pl.ds(base, ROWS_LO)],
                        idx_d.at[pl.ds(0, ROWS_LO)])

        @pl.when(count > ROWS_LO)
        def _():
            pltpu.sync_copy(ei_hbm.at[1, pl.ds(base + ROWS_LO, 1)],
                            idx_d.at[pl.ds(ROWS_LO, 1)])

        pltpu.sync_copy(ones_hbm, rows)
        plsc.subcore_barrier()

        def body(e, carry):
            pltpu.sync_copy(rows, acc.at[idx_d.at[e]], add=True)
            return carry

        lax.fori_loop(0, count, body, 0)
        plsc.subcore_barrier()
        pltpu.sync_copy(acc.at[pl.ds(r0, ROWS_PER_TILE)],
                        out_hbm.at[cid, pl.ds(r0, ROWS_PER_TILE)])

    return deg


_deg = _make_deg()

_ROW_BLK = 1000
_GRID = N_NODES // _ROW_BLK


def _tca_body(x_ref, w1_ref, xw_ref):
    xw_ref[...] = jnp.dot(x_ref[...], w1_ref[...],
                          preferred_element_type=jnp.float32)


def _tcb_body(xw_ref, p_ref, y1_ref, dinv_ref):
    deg = p_ref[0, :, 0:1] + p_ref[1, :, 0:1] + 1.0
    dinv = lax.rsqrt(deg)
    y1_ref[...] = xw_ref[...] * dinv
    dinv_ref[...] = dinv


def _tc2_body(p_ref, y1_ref, dinv_ref, b1_ref, w2_ref, y2_ref):
    dinv = dinv_ref[...]
    h = (p_ref[0] + p_ref[1] + y1_ref[...]) * dinv + b1_ref[...]
    h = jnp.maximum(h, 0.0)
    y2_ref[...] = jnp.dot(h, w2_ref[...], preferred_element_type=jnp.float32) * dinv


def _tc3_body(q_ref, y2_ref, dinv_ref, b2_ref, out_ref):
    o = (q_ref[0] + q_ref[1] + y2_ref[...]) * dinv_ref[...] + b2_ref[...]
    m = jnp.max(o, axis=1, keepdims=True)
    out_ref[...] = o - m - jnp.log(jnp.sum(jnp.exp(o - m), axis=1, keepdims=True))


def _row_spec(d):
    return pl.BlockSpec((_ROW_BLK, d), lambda i: (i, 0))


def _pair_spec(d):
    return pl.BlockSpec((NC, _ROW_BLK, d), lambda i: (0, i, 0))


def _full_spec(r, c):
    return pl.BlockSpec((r, c), lambda i: (0, 0))


def kernel(x, edge_index, W1, b1, W2, b2):
    ei = edge_index.astype(jnp.int32).reshape(2, N_ROWS, BATCH)
    ones_tile = jnp.ones((BATCH, _DEG_D), jnp.float32)
    zeros16 = jnp.zeros((N_PAD, 16), jnp.float32)
    zeros64 = jnp.zeros((N_PAD, 64), jnp.float32)
    b1r = b1.reshape(1, -1)
    b2r = b2.reshape(1, -1)

    pdeg = _deg(ones_tile, ei, zeros16)

    xw = pl.pallas_call(
        _tca_body,
        grid=(_GRID,),
        in_specs=[_row_spec(128), _full_spec(128, 64)],
        out_specs=_row_spec(64),
        out_shape=jax.ShapeDtypeStruct((N_NODES, 64), jnp.float32),
    )(x, W1)

    y1, dinv = pl.pallas_call(
        _tcb_body,
        grid=(_GRID,),
        in_specs=[_row_spec(64), _pair_spec(16)],
        out_specs=[_row_spec(64), _row_spec(1)],
        out_shape=[
            jax.ShapeDtypeStruct((N_NODES, 64), jnp.float32),
            jax.ShapeDtypeStruct((N_NODES, 1), jnp.float32),
        ],
    )(xw, pdeg)

    p = _agg64(y1, ei, zeros64)

    y2 = pl.pallas_call(
        _tc2_body,
        grid=(_GRID,),
        in_specs=[_pair_spec(64), _row_spec(64), _row_spec(1),
                  _full_spec(1, 64), _full_spec(64, 16)],
        out_specs=_row_spec(16),
        out_shape=jax.ShapeDtypeStruct((N_NODES, 16), jnp.float32),
    )(p, y1, dinv, b1r, W2)

    q = _agg16(y2, ei, zeros16)

    out = pl.pallas_call(
        _tc3_body,
        grid=(_GRID,),
        in_specs=[_pair_spec(16), _row_spec(16), _row_spec(1),
                  _full_spec(1, 16)],
        out_specs=_row_spec(16),
        out_shape=jax.ShapeDtypeStruct((N_NODES, 16), jnp.float32),
    )(q, y2, dinv, b2r)
    return out


# speedup vs baseline: 1.0063x; 1.0063x over previous
import functools

import jax
import jax.numpy as jnp
from jax import lax
from jax.experimental import pallas as pl
from jax.experimental.pallas import tpu as pltpu
from jax.experimental.pallas import tpu_sc as plsc

N_NODES = 10000
N_EDGES = 320000
NC = 2
NS = 16
NW = NC * NS
BATCH = 128
N_ROWS = N_EDGES // BATCH
ROWS_LO = N_ROWS // NW
N_HI = N_ROWS - ROWS_LO * NW
ROWS_MAX = ROWS_LO + 1
N_PAD = 10240
ROWS_PER_TILE = N_PAD // NS


def _worker_rows(wid):
    base = ROWS_LO * wid + jnp.minimum(wid, N_HI)
    count = ROWS_LO + (wid < N_HI).astype(jnp.int32)
    return base, count


def _make_agg(D, depth):
    mesh = plsc.VectorSubcoreMesh(core_axis_name="c", subcore_axis_name="s")

    @functools.partial(
        pl.kernel,
        mesh=mesh,
        compiler_params=pltpu.CompilerParams(use_tc_tiling_on_sc=False),
        out_type=jax.ShapeDtypeStruct((NC, N_PAD, D), jnp.float32),
        scratch_types=(
            [
                pltpu.VMEM((ROWS_MAX, BATCH), jnp.int32),
                pltpu.VMEM((ROWS_MAX, BATCH), jnp.int32),
            ]
            + [pltpu.VMEM((BATCH, D), jnp.float32)] * depth
            + [pltpu.VMEM_SHARED((N_PAD, D), jnp.float32)]
            + [pltpu.SemaphoreType.DMA] * depth
        ),
    )
    def agg(table_hbm, ei_hbm, zeros_hbm, out_hbm, *scratch):
        idx_s, idx_d = scratch[0], scratch[1]
        bufs = scratch[2:2 + depth]
        acc = scratch[2 + depth]
        sems = scratch[3 + depth:]
        cid = lax.axis_index("c")
        sid = lax.axis_index("s")
        wid = sid * NC + cid
        r0 = sid * ROWS_PER_TILE
        pltpu.sync_copy(zeros_hbm.at[pl.ds(r0, ROWS_PER_TILE)],
                        acc.at[pl.ds(r0, ROWS_PER_TILE)])
        base, count = _worker_rows(wid)
        pltpu.sync_copy(ei_hbm.at[0, pl.ds(base, ROWS_LO)],
                        idx_s.at[pl.ds(0, ROWS_LO)])
        pltpu.sync_copy(ei_hbm.at[1, pl.ds(base, ROWS_LO)],
                        idx_d.at[pl.ds(0, ROWS_LO)])

        @pl.when(count > ROWS_LO)
        def _():
            pltpu.sync_copy(ei_hbm.at[0, pl.ds(base + ROWS_LO, 1)],
                            idx_s.at[pl.ds(ROWS_LO, 1)])
            pltpu.sync_copy(ei_hbm.at[1, pl.ds(base + ROWS_LO, 1)],
                            idx_d.at[pl.ds(ROWS_LO, 1)])

        plsc.subcore_barrier()

        for j in range(depth):
            pltpu.async_copy(table_hbm.at[idx_s.at[j]], bufs[j], sems[j])

        def body(i, carry):
            e = depth * i
            for j in range(depth):
                pltpu.make_async_copy(
                    table_hbm.at[idx_s.at[0]], bufs[j], sems[j]).wait()
                pltpu.sync_copy(bufs[j], acc.at[idx_d.at[e + j]], add=True)
                nxt = lax.rem(e + j + depth, count)
                pltpu.async_copy(table_hbm.at[idx_s.at[nxt]], bufs[j], sems[j])
            return carry

        lax.fori_loop(0, ROWS_LO // depth, body, 0)

        e0 = (ROWS_LO // depth) * depth
        for j in range(depth):
            pltpu.make_async_copy(
                table_hbm.at[idx_s.at[0]], bufs[j], sems[j]).wait()
            e = e0 + j
            if e < ROWS_LO:
                pltpu.sync_copy(bufs[j], acc.at[idx_d.at[e]], add=True)
            elif e == ROWS_LO:
                @pl.when(count > ROWS_LO)
                def _(buf=bufs[j]):
                    pltpu.sync_copy(buf, acc.at[idx_d.at[ROWS_LO]], add=True)

        plsc.subcore_barrier()
        pltpu.sync_copy(acc.at[pl.ds(r0, ROWS_PER_TILE)],
                        out_hbm.at[cid, pl.ds(r0, ROWS_PER_TILE)])

    return agg


_agg16 = _make_agg(16, 8)
_agg64 = _make_agg(64, 4)

_DEG_D = 16


def _make_deg():
    mesh = plsc.VectorSubcoreMesh(core_axis_name="c", subcore_axis_name="s")

    @functools.partial(
        pl.kernel,
        mesh=mesh,
        compiler_params=pltpu.CompilerParams(use_tc_tiling_on_sc=False),
        out_type=jax.ShapeDtypeStruct((NC, N_PAD, _DEG_D), jnp.float32),
        scratch_types=[
            pltpu.VMEM((ROWS_MAX, BATCH), jnp.int32),
            pltpu.VMEM((BATCH, _DEG_D), jnp.float32),
            pltpu.VMEM_SHARED((N_PAD, _DEG_D), jnp.float32),
        ],
    )
    def deg(ones_hbm, ei_hbm, zeros_hbm, out_hbm, idx_d, rows, acc):
        cid = lax.axis_index("c")
        sid = lax.axis_index("s")
        wid = sid * NC + cid
        r0 = sid * ROWS_PER_TILE
        pltpu.sync_copy(zeros_hbm.at[pl.ds(r0, ROWS_PER_TILE)],
                        acc.at[pl.ds(r0, ROWS_PER_TILE)])
        base, count = _worker_rows(wid)
        pltpu.sync_copy(ei_hbm.at[1, pl.ds(base, ROWS_LO)],
                        idx_d.at[pl.ds(0, ROWS_LO)])

        @pl.when(count > ROWS_LO)
        def _():
            pltpu.sync_copy(ei_hbm.at[1, pl.ds(base + ROWS_LO, 1)],
                            idx_d.at[pl.ds(ROWS_LO, 1)])

        pltpu.sync_copy(ones_hbm, rows)
        plsc.subcore_barrier()

        def body(e, carry):
            pltpu.sync_copy(rows, acc.at[idx_d.at[e]], add=True)
            return carry

        lax.fori_loop(0, count, body, 0)
        plsc.subcore_barrier()
        pltpu.sync_copy(acc.at[pl.ds(r0, ROWS_PER_TILE)],
                        out_hbm.at[cid, pl.ds(r0, ROWS_PER_TILE)])

    return deg


_deg = _make_deg()

_ROW_BLK = 1000
_GRID = N_NODES // _ROW_BLK


def _tcb_body(x_ref, w1_ref, p_ref, y1_ref, dinv_ref):
    deg = p_ref[0, :, 0:1] + p_ref[1, :, 0:1] + 1.0
    dinv = lax.rsqrt(deg)
    xw = jnp.dot(x_ref[...], w1_ref[...], preferred_element_type=jnp.float32)
    y1_ref[...] = xw * dinv
    dinv_ref[...] = dinv


def _tc2_body(p_ref, y1_ref, dinv_ref, b1_ref, w2_ref, y2_ref):
    dinv = dinv_ref[...]
    h = (p_ref[0] + p_ref[1] + y1_ref[...]) * dinv + b1_ref[...]
    h = jnp.maximum(h, 0.0)
    y2_ref[...] = jnp.dot(h, w2_ref[...], preferred_element_type=jnp.float32) * dinv


def _tc3_body(q_ref, y2_ref, dinv_ref, b2_ref, out_ref):
    o = (q_ref[0] + q_ref[1] + y2_ref[...]) * dinv_ref[...] + b2_ref[...]
    m = jnp.max(o, axis=1, keepdims=True)
    out_ref[...] = o - m - jnp.log(jnp.sum(jnp.exp(o - m), axis=1, keepdims=True))


def _row_spec(d):
    return pl.BlockSpec((_ROW_BLK, d), lambda i: (i, 0))


def _pair_spec(d):
    return pl.BlockSpec((NC, _ROW_BLK, d), lambda i: (0, i, 0))


def _full_spec(r, c):
    return pl.BlockSpec((r, c), lambda i: (0, 0))


def kernel(x, edge_index, W1, b1, W2, b2):
    ei = edge_index.astype(jnp.int32).reshape(2, N_ROWS, BATCH)
    ones_tile = jnp.ones((BATCH, _DEG_D), jnp.float32)
    zeros16 = jnp.zeros((N_PAD, 16), jnp.float32)
    zeros64 = jnp.zeros((N_PAD, 64), jnp.float32)
    b1r = b1.reshape(1, -1)
    b2r = b2.reshape(1, -1)

    pdeg = _deg(ones_tile, ei, zeros16)

    y1, dinv = pl.pallas_call(
        _tcb_body,
        grid=(_GRID,),
        in_specs=[_row_spec(128), _full_spec(128, 64), _pair_spec(16)],
        out_specs=[_row_spec(64), _row_spec(1)],
        out_shape=[
            jax.ShapeDtypeStruct((N_NODES, 64), jnp.float32),
            jax.ShapeDtypeStruct((N_NODES, 1), jnp.float32),
        ],
    )(x, W1, pdeg)

    p = _agg64(y1, ei, zeros64)

    y2 = pl.pallas_call(
        _tc2_body,
        grid=(_GRID,),
        in_specs=[_pair_spec(64), _row_spec(64), _row_spec(1),
                  _full_spec(1, 64), _full_spec(64, 16)],
        out_specs=_row_spec(16),
        out_shape=jax.ShapeDtypeStruct((N_NODES, 16), jnp.float32),
    )(p, y1, dinv, b1r, W2)

    q = _agg16(y2, ei, zeros16)

    out = pl.pallas_call(
        _tc3_body,
        grid=(_GRID,),
        in_specs=[_pair_spec(16), _row_spec(16), _row_spec(1),
                  _full_spec(1, 16)],
        out_specs=_row_spec(16),
        out_shape=jax.ShapeDtypeStruct((N_NODES, 16), jnp.float32),
    )(q, y2, dinv, b2r)
    return out


# speedup vs baseline: 1.0380x; 1.0314x over previous
import functools

import jax
import jax.numpy as jnp
from jax import lax
from jax.experimental import pallas as pl
from jax.experimental.pallas import tpu as pltpu
from jax.experimental.pallas import tpu_sc as plsc

N_NODES = 10000
N_EDGES = 320000
NC = 2
NS = 16
NW = NC * NS
BATCH = 128
N_ROWS = N_EDGES // BATCH
ROWS_LO = N_ROWS // NW
N_HI = N_ROWS - ROWS_LO * NW
ROWS_MAX = ROWS_LO + 1
N_PAD = 10240
ROWS_PER_TILE = N_PAD // NS


def _worker_rows(wid):
    base = ROWS_LO * wid + jnp.minimum(wid, N_HI)
    count = ROWS_LO + (wid < N_HI).astype(jnp.int32)
    return base, count


def _make_agg(D, depth):
    mesh = plsc.VectorSubcoreMesh(core_axis_name="c", subcore_axis_name="s")

    @functools.partial(
        pl.kernel,
        mesh=mesh,
        compiler_params=pltpu.CompilerParams(use_tc_tiling_on_sc=False),
        out_type=jax.ShapeDtypeStruct((NC, N_PAD, D), jnp.float32),
        scratch_types=(
            [
                pltpu.VMEM((ROWS_MAX, BATCH), jnp.int32),
                pltpu.VMEM((ROWS_MAX, BATCH), jnp.int32),
            ]
            + [pltpu.VMEM((BATCH, D), jnp.float32)] * depth
            + [pltpu.VMEM_SHARED((N_PAD, D), jnp.float32)]
            + [pltpu.SemaphoreType.DMA] * depth
        ),
    )
    def agg(table_hbm, ei_hbm, zeros_hbm, out_hbm, *scratch):
        idx_s, idx_d = scratch[0], scratch[1]
        bufs = scratch[2:2 + depth]
        acc = scratch[2 + depth]
        sems = scratch[3 + depth:]
        cid = lax.axis_index("c")
        sid = lax.axis_index("s")
        wid = sid * NC + cid
        r0 = sid * ROWS_PER_TILE
        pltpu.sync_copy(zeros_hbm.at[pl.ds(r0, ROWS_PER_TILE)],
                        acc.at[pl.ds(r0, ROWS_PER_TILE)])
        base, count = _worker_rows(wid)
        pltpu.sync_copy(ei_hbm.at[0, pl.ds(base, ROWS_LO)],
                        idx_s.at[pl.ds(0, ROWS_LO)])
        pltpu.sync_copy(ei_hbm.at[1, pl.ds(base, ROWS_LO)],
                        idx_d.at[pl.ds(0, ROWS_LO)])

        @pl.when(count > ROWS_LO)
        def _():
            pltpu.sync_copy(ei_hbm.at[0, pl.ds(base + ROWS_LO, 1)],
                            idx_s.at[pl.ds(ROWS_LO, 1)])
            pltpu.sync_copy(ei_hbm.at[1, pl.ds(base + ROWS_LO, 1)],
                            idx_d.at[pl.ds(ROWS_LO, 1)])

        plsc.subcore_barrier()

        for j in range(depth):
            pltpu.async_copy(table_hbm.at[idx_s.at[j]], bufs[j], sems[j])

        def body(i, carry):
            e = depth * i
            for j in range(depth):
                pltpu.make_async_copy(
                    table_hbm.at[idx_s.at[0]], bufs[j], sems[j]).wait()
                pltpu.sync_copy(bufs[j], acc.at[idx_d.at[e + j]], add=True)
                nxt = lax.rem(e + j + depth, count)
                pltpu.async_copy(table_hbm.at[idx_s.at[nxt]], bufs[j], sems[j])
            return carry

        lax.fori_loop(0, ROWS_LO // depth, body, 0)

        e0 = (ROWS_LO // depth) * depth
        for j in range(depth):
            pltpu.make_async_copy(
                table_hbm.at[idx_s.at[0]], bufs[j], sems[j]).wait()
            e = e0 + j
            if e < ROWS_LO:
                pltpu.sync_copy(bufs[j], acc.at[idx_d.at[e]], add=True)
            elif e == ROWS_LO:
                @pl.when(count > ROWS_LO)
                def _(buf=bufs[j]):
                    pltpu.sync_copy(buf, acc.at[idx_d.at[ROWS_LO]], add=True)

        plsc.subcore_barrier()
        pltpu.sync_copy(acc.at[pl.ds(r0, ROWS_PER_TILE)],
                        out_hbm.at[cid, pl.ds(r0, ROWS_PER_TILE)])

    return agg


_agg16 = _make_agg(16, 8)
_agg64 = _make_agg(64, 4)

_DEG_D = 16


def _make_deg():
    mesh = plsc.VectorSubcoreMesh(core_axis_name="c", subcore_axis_name="s")

    @functools.partial(
        pl.kernel,
        mesh=mesh,
        compiler_params=pltpu.CompilerParams(use_tc_tiling_on_sc=False),
        out_type=jax.ShapeDtypeStruct((NC, N_PAD, _DEG_D), jnp.float32),
        scratch_types=[
            pltpu.VMEM((ROWS_MAX, BATCH), jnp.int32),
            pltpu.VMEM((BATCH, _DEG_D), jnp.float32),
            pltpu.VMEM_SHARED((N_PAD, _DEG_D), jnp.float32),
        ],
    )
    def deg(ones_hbm, ei_hbm, zeros_hbm, out_hbm, idx_d, rows, acc):
        cid = lax.axis_index("c")
        sid = lax.axis_index("s")
        wid = sid * NC + cid
        r0 = sid * ROWS_PER_TILE
        pltpu.sync_copy(zeros_hbm.at[pl.ds(r0, ROWS_PER_TILE)],
                        acc.at[pl.ds(r0, ROWS_PER_TILE)])
        base, count = _worker_rows(wid)
        pltpu.sync_copy(ei_hbm.at[1, pl.ds(base, ROWS_LO)],
                        idx_d.at[pl.ds(0, ROWS_LO)])

        @pl.when(count > ROWS_LO)
        def _():
            pltpu.sync_copy(ei_hbm.at[1, pl.ds(base + ROWS_LO, 1)],
                            idx_d.at[pl.ds(ROWS_LO, 1)])

        pltpu.sync_copy(ones_hbm, rows)
        plsc.subcore_barrier()

        def body(e, carry):
            pltpu.sync_copy(rows, acc.at[idx_d.at[e]], add=True)
            return carry

        lax.fori_loop(0, count, body, 0)
        plsc.subcore_barrier()
        pltpu.sync_copy(acc.at[pl.ds(r0, ROWS_PER_TILE)],
                        out_hbm.at[cid, pl.ds(r0, ROWS_PER_TILE)])

    return deg


_deg = _make_deg()

_ROW_BLK = 2000
_GRID = N_NODES // _ROW_BLK


def _tcb_body(x_ref, w1_ref, p_ref, y1_ref, dinv_ref):
    deg = p_ref[0, :, 0:1] + p_ref[1, :, 0:1] + 1.0
    dinv = lax.rsqrt(deg)
    xw = jnp.dot(x_ref[...], w1_ref[...], preferred_element_type=jnp.float32)
    y1_ref[...] = xw * dinv
    dinv_ref[...] = dinv


def _tc2_body(p_ref, y1_ref, dinv_ref, b1_ref, w2_ref, y2_ref):
    dinv = dinv_ref[...]
    h = (p_ref[0] + p_ref[1] + y1_ref[...]) * dinv + b1_ref[...]
    h = jnp.maximum(h, 0.0)
    y2_ref[...] = jnp.dot(h, w2_ref[...], preferred_element_type=jnp.float32) * dinv


def _tc3_body(q_ref, y2_ref, dinv_ref, b2_ref, out_ref):
    o = (q_ref[0] + q_ref[1] + y2_ref[...]) * dinv_ref[...] + b2_ref[...]
    m = jnp.max(o, axis=1, keepdims=True)
    out_ref[...] = o - m - jnp.log(jnp.sum(jnp.exp(o - m), axis=1, keepdims=True))


def _row_spec(d):
    return pl.BlockSpec((_ROW_BLK, d), lambda i: (i, 0))


def _pair_spec(d):
    return pl.BlockSpec((NC, _ROW_BLK, d), lambda i: (0, i, 0))


def _full_spec(r, c):
    return pl.BlockSpec((r, c), lambda i: (0, 0))


def kernel(x, edge_index, W1, b1, W2, b2):
    ei = edge_index.astype(jnp.int32).reshape(2, N_ROWS, BATCH)
    ones_tile = jnp.ones((BATCH, _DEG_D), jnp.float32)
    zeros16 = jnp.zeros((N_PAD, 16), jnp.float32)
    zeros64 = jnp.zeros((N_PAD, 64), jnp.float32)
    b1r = b1.reshape(1, -1)
    b2r = b2.reshape(1, -1)

    pdeg = _deg(ones_tile, ei, zeros16)

    y1, dinv = pl.pallas_call(
        _tcb_body,
        grid=(_GRID,),
        in_specs=[_row_spec(128), _full_spec(128, 64), _pair_spec(16)],
        out_specs=[_row_spec(64), _row_spec(1)],
        out_shape=[
            jax.ShapeDtypeStruct((N_NODES, 64), jnp.float32),
            jax.ShapeDtypeStruct((N_NODES, 1), jnp.float32),
        ],
    )(x, W1, pdeg)

    p = _agg64(y1, ei, zeros64)

    y2 = pl.pallas_call(
        _tc2_body,
        grid=(_GRID,),
        in_specs=[_pair_spec(64), _row_spec(64), _row_spec(1),
                  _full_spec(1, 64), _full_spec(64, 16)],
        out_specs=_row_spec(16),
        out_shape=jax.ShapeDtypeStruct((N_NODES, 16), jnp.float32),
    )(p, y1, dinv, b1r, W2)

    q = _agg16(y2, ei, zeros16)

    out = pl.pallas_call(
        _tc3_body,
        grid=(_GRID,),
        in_specs=[_pair_spec(16), _row_spec(16), _row_spec(1),
                  _full_spec(1, 16)],
        out_specs=_row_spec(16),
        out_shape=jax.ShapeDtypeStruct((N_NODES, 16), jnp.float32),
    )(q, y2, dinv, b2r)
    return out
